# baseline (speedup 1.0000x reference)
import functools

import jax
import jax.numpy as jnp
from jax import lax
from jax.experimental import pallas as pl
from jax.experimental.pallas import tpu as pltpu
from jax.experimental.pallas import tpu_sc as plsc

N = 10000
E = 320000
D = 128
G = 64

NP = 10240
BLK = 512
NBLK = NP // BLK

CHUNK = 80
EP = 322560
NCH = EP // 32 // CHUNK
ETILE = EP // 32
ROWS_T = NP // 16

_HI = lax.Precision.HIGHEST
_DN = (((1,), (1,)), ((), ()))
_DNT = (((1,), (0,)), ((), ()))


def _stage_first_body(h_ref, wrel_ref, wroot_ref, brel_ref, y_ref, r_ref):
    h = h_ref[...]
    y_ref[...] = lax.dot_general(h, wrel_ref[...], _DN, precision=_HI)
    r_ref[...] = lax.dot_general(h, wroot_ref[...], _DN, precision=_HI) + brel_ref[...]


def _stage_body(add_h, agga_ref, aggb_ref, rp_ref, wrel_ref, wroot_ref, brel_ref,
                y_ref, r_ref):
    h = jnp.maximum(agga_ref[...] + aggb_ref[...] + rp_ref[...], 0.0)
    y_ref[...] = lax.dot_general(h, wrel_ref[...], _DN, precision=_HI)
    r = lax.dot_general(h, wroot_ref[...], _DN, precision=_HI) + brel_ref[...]
    if add_h:
        r = r + h
    r_ref[...] = r


_ROW_SPEC = pl.BlockSpec((BLK, D), lambda i: (i, 0))
_W_SPEC = pl.BlockSpec((D, D), lambda i: (0, 0))
_B_SPEC = pl.BlockSpec((1, D), lambda i: (0, 0))


def _tc_stage_first(x, wrel, wroot, brel):
    return pl.pallas_call(
        _stage_first_body,
        grid=(NBLK,),
        in_specs=[_ROW_SPEC, _W_SPEC, _W_SPEC, _B_SPEC],
        out_specs=[_ROW_SPEC, _ROW_SPEC],
        out_shape=[jax.ShapeDtypeStruct((NP, D), jnp.float32)] * 2,
    )(x, wrel, wroot, brel.reshape(1, D))


def _tc_stage(agg_a, agg_b, r_prev, wrel, wroot, brel, add_h):
    return pl.pallas_call(
        functools.partial(_stage_body, add_h),
        grid=(NBLK,),
        in_specs=[_ROW_SPEC, _ROW_SPEC, _ROW_SPEC, _W_SPEC, _W_SPEC, _B_SPEC],
        out_specs=[_ROW_SPEC, _ROW_SPEC],
        out_shape=[jax.ShapeDtypeStruct((NP, D), jnp.float32)] * 2,
    )(agg_a, agg_b, r_prev, wrel, wroot, brel.reshape(1, D))


def _final_body(agga_ref, aggb_ref, rp_ref, batch_ref, wlin_ref, blin_ref,
                out_ref, sums_ref, cnts_ref):
    i = pl.program_id(0)

    @pl.when(i == 0)
    def _():
        sums_ref[...] = jnp.zeros_like(sums_ref)
        cnts_ref[...] = jnp.zeros_like(cnts_ref)

    h = jnp.maximum(agga_ref[...] + aggb_ref[...] + rp_ref[...], 0.0)
    b = batch_ref[0]
    gids = lax.broadcasted_iota(jnp.int32, (G, BLK), 0)
    m = (b == gids).astype(jnp.float32)
    sums_ref[...] += lax.dot_general(m, h, _DNT, precision=_HI)
    cnts_ref[...] += lax.dot_general(m, jnp.ones((BLK, D), jnp.float32), _DNT,
                                     precision=_HI)

    @pl.when(i == NBLK - 1)
    def _():
        pooled = sums_ref[...] / jnp.maximum(cnts_ref[...], 1.0)
        logits = lax.dot_general(pooled, wlin_ref[...], _DN, precision=_HI)
        logits = logits + blin_ref[...]
        mx = jnp.max(logits, axis=1, keepdims=True)
        e = jnp.exp(logits - mx)
        out_ref[...] = e / jnp.sum(e, axis=1, keepdims=True)


def _tc_final(agg_a, agg_b, r_prev, batch3, wlin, blin):
    return pl.pallas_call(
        _final_body,
        grid=(NBLK,),
        in_specs=[_ROW_SPEC, _ROW_SPEC, _ROW_SPEC,
                  pl.BlockSpec((1, 1, BLK), lambda i: (i, 0, 0)),
                  _W_SPEC, _B_SPEC],
        out_specs=pl.BlockSpec((G, D), lambda i: (0, 0)),
        out_shape=jax.ShapeDtypeStruct((G, D), jnp.float32),
        scratch_shapes=[pltpu.VMEM((G, D), jnp.float32),
                        pltpu.VMEM((G, D), jnp.float32)],
    )(agg_a, agg_b, r_prev, batch3, wlin, blin.reshape(1, D))


def _sc_segsum_body(y_hbm, src_hbm, dst_hbm, zeros_hbm, out_a, out_b,
                    src_cb, dst_b0, dst_b1, dst_b2, dst_b3, rows_v, agg_sh,
                    isem0, isem1, isem2, isem3,
                    jsem0, jsem1, jsem2, jsem3, gsem0, gsem1, semz):
    cid = lax.axis_index("c")
    sid = lax.axis_index("s")
    row0 = sid * ROWS_T

    zcopy = pltpu.async_copy(zeros_hbm.at[pl.ds(row0, ROWS_T)],
                             agg_sh.at[pl.ds(row0, ROWS_T)], semz)

    base = (cid * 16 + sid) * ETILE

    isems = (isem0, isem1, isem2, isem3)
    jsems = (jsem0, jsem1, jsem2, jsem3)
    gsems = (gsem0, gsem1)
    dstrefs = (dst_b0, dst_b1, dst_b2, dst_b3)

    def _fire_idx(c, sl):
        off = base + jnp.minimum(c, NCH - 1) * CHUNK
        pltpu.async_copy(src_hbm.at[pl.ds(off, CHUNK)], src_cb.at[sl],
                         isems[sl])
        pltpu.async_copy(dst_hbm.at[pl.ds(off, CHUNK)], dstrefs[sl],
                         jsems[sl])

    def _wait_idx(sl):
        pltpu.make_async_copy(src_hbm.at[pl.ds(base, CHUNK)], src_cb.at[sl],
                              isems[sl]).wait()
        pltpu.make_async_copy(dst_hbm.at[pl.ds(base, CHUNK)], dstrefs[sl],
                              jsems[sl]).wait()

    def _fire_gather(sl, rb):
        pltpu.async_copy(y_hbm.at[src_cb.at[sl]], rows_v.at[rb], gsems[rb])

    def _wait_gather(sl, rb):
        pltpu.make_async_copy(y_hbm.at[src_cb.at[sl]], rows_v.at[rb],
                              gsems[rb]).wait()

    for sl in range(4):
        _fire_idx(sl, sl)
    for c in range(2):
        _wait_idx(c)
        _fire_gather(c, c)
    zcopy.wait()
    plsc.subcore_barrier()

    def _body(p, carry):
        for u in range(4):
            c = 4 * p + u
            rb = u % 2
            _wait_gather(u, rb)
            pltpu.sync_copy(rows_v.at[rb], agg_sh.at[dstrefs[u]], add=True)
            _fire_idx(c + 4, u)
            _wait_idx((u + 2) % 4)
            _fire_gather((u + 2) % 4, rb)
        return carry

    lax.fori_loop(0, (NCH - 2) // 4, _body, 0)
    for u in range(2):
        rb = u % 2
        _wait_gather(u, rb)
        pltpu.sync_copy(rows_v.at[rb], agg_sh.at[dstrefs[u]], add=True)
        _wait_idx(u + 2)
    plsc.subcore_barrier()

    @pl.when(cid == 0)
    def _():
        pltpu.sync_copy(agg_sh.at[pl.ds(row0, ROWS_T)],
                        out_a.at[pl.ds(row0, ROWS_T)])

    @pl.when(cid == 1)
    def _():
        pltpu.sync_copy(agg_sh.at[pl.ds(row0, ROWS_T)],
                        out_b.at[pl.ds(row0, ROWS_T)])


@functools.lru_cache(maxsize=1)
def _get_sc_segsum():
    return pl.kernel(
        _sc_segsum_body,
        out_type=[jax.ShapeDtypeStruct((NP, D), jnp.float32)] * 2,
        mesh=plsc.VectorSubcoreMesh(core_axis_name="c", subcore_axis_name="s"),
        scratch_types=[
            pltpu.VMEM((4, CHUNK), jnp.int32),
            pltpu.VMEM((CHUNK,), jnp.int32),
            pltpu.VMEM((CHUNK,), jnp.int32),
            pltpu.VMEM((CHUNK,), jnp.int32),
            pltpu.VMEM((CHUNK,), jnp.int32),
            pltpu.VMEM((2, CHUNK, D), jnp.float32),
            pltpu.VMEM_SHARED((NP, D), jnp.float32),
        ] + [pltpu.SemaphoreType.DMA] * 11,
    )


def _sc_segsum(y, src, dst, zeros):
    return _get_sc_segsum()(y, src, dst, zeros)


def kernel(x, edge_index, batch,
           Wrel0, brel0, Wroot0,
           Wrel1, brel1, Wroot1,
           Wrel2, brel2, Wroot2,
           Wrel3, brel3, Wroot3,
           Wrel4, brel4, Wroot4,
           Wlin, blin):
    xp = jnp.zeros((NP, D), jnp.float32).at[:N].set(x)
    npad = EP - E
    src1 = jnp.concatenate([edge_index[0], jnp.zeros((npad,), jnp.int32)])
    dst1 = jnp.concatenate(
        [edge_index[1], N + (jnp.arange(npad, dtype=jnp.int32) % (NP - N))])
    zeros = jnp.zeros((NP, D), jnp.float32)
    batch3 = jnp.full((NP,), G, jnp.int32).at[:N].set(batch).reshape(NBLK, 1, BLK)

    y, r = _tc_stage_first(xp, Wrel0, Wroot0, brel0)
    for wrel, wroot, brel, add_h in (
            (Wrel1, Wroot1, brel1, False),
            (Wrel2, Wroot2, brel2, False),
            (Wrel3, Wroot3, brel3, True),
            (Wrel4, Wroot4, brel4, True)):
        agg_a, agg_b = _sc_segsum(y, src1, dst1, zeros)
        y, r = _tc_stage(agg_a, agg_b, r, wrel, wroot, brel, add_h)
    agg_a, agg_b = _sc_segsum(y, src1, dst1, zeros)
    return _tc_final(agg_a, agg_b, r, batch3, Wlin, blin)


# speedup vs baseline: 1.8888x; 1.8888x over previous
import functools

import jax
import jax.numpy as jnp
from jax import lax
from jax.experimental import pallas as pl
from jax.experimental.pallas import tpu as pltpu
from jax.experimental.pallas import tpu_sc as plsc

N = 10000
E = 320000
D = 128
G = 64

NP = 10240
BLK = 512
NBLK = NP // BLK

CHUNK = 80
EP = 322560
NCH = EP // 32 // CHUNK
ETILE = EP // 32
ROWS_T = NP // 16

_HI = lax.Precision.HIGHEST
_DN = (((1,), (1,)), ((), ()))
_DNT = (((1,), (0,)), ((), ()))


def _stage_first_body(h_ref, wrel_ref, wroot_ref, brel_ref, y_ref, r_ref):
    h = h_ref[...]
    y_ref[...] = lax.dot_general(h, wrel_ref[...], _DN, precision=_HI)
    r_ref[...] = lax.dot_general(h, wroot_ref[...], _DN, precision=_HI) + brel_ref[...]


def _stage_body(add_h, agga_ref, aggb_ref, rp_ref, wrel_ref, wroot_ref, brel_ref,
                y_ref, r_ref):
    h = jnp.maximum(agga_ref[...] + aggb_ref[...] + rp_ref[...], 0.0)
    y_ref[...] = lax.dot_general(h, wrel_ref[...], _DN, precision=_HI)
    r = lax.dot_general(h, wroot_ref[...], _DN, precision=_HI) + brel_ref[...]
    if add_h:
        r = r + h
    r_ref[...] = r


_ROW_SPEC = pl.BlockSpec((BLK, D), lambda i: (i, 0))
_W_SPEC = pl.BlockSpec((D, D), lambda i: (0, 0))
_B_SPEC = pl.BlockSpec((1, D), lambda i: (0, 0))


def _tc_stage_first(x, wrel, wroot, brel):
    return pl.pallas_call(
        _stage_first_body,
        grid=(NBLK,),
        in_specs=[_ROW_SPEC, _W_SPEC, _W_SPEC, _B_SPEC],
        out_specs=[_ROW_SPEC, _ROW_SPEC],
        out_shape=[jax.ShapeDtypeStruct((NP, D), jnp.float32)] * 2,
    )(x, wrel, wroot, brel.reshape(1, D))


def _tc_stage(agg_a, agg_b, r_prev, wrel, wroot, brel, add_h):
    return pl.pallas_call(
        functools.partial(_stage_body, add_h),
        grid=(NBLK,),
        in_specs=[_ROW_SPEC, _ROW_SPEC, _ROW_SPEC, _W_SPEC, _W_SPEC, _B_SPEC],
        out_specs=[_ROW_SPEC, _ROW_SPEC],
        out_shape=[jax.ShapeDtypeStruct((NP, D), jnp.float32)] * 2,
    )(agg_a, agg_b, r_prev, wrel, wroot, brel.reshape(1, D))


def _final_body(agga_ref, aggb_ref, rp_ref, batch_ref, wlin_ref, blin_ref,
                out_ref, sums_ref, cnts_ref):
    i = pl.program_id(0)

    @pl.when(i == 0)
    def _():
        sums_ref[...] = jnp.zeros_like(sums_ref)
        cnts_ref[...] = jnp.zeros_like(cnts_ref)

    h = jnp.maximum(agga_ref[...] + aggb_ref[...] + rp_ref[...], 0.0)
    b = batch_ref[0]
    gids = lax.broadcasted_iota(jnp.int32, (G, BLK), 0)
    m = (b == gids).astype(jnp.float32)
    sums_ref[...] += lax.dot_general(m, h, _DNT, precision=_HI)
    cnts_ref[...] += lax.dot_general(m, jnp.ones((BLK, D), jnp.float32), _DNT,
                                     precision=_HI)

    @pl.when(i == NBLK - 1)
    def _():
        pooled = sums_ref[...] / jnp.maximum(cnts_ref[...], 1.0)
        logits = lax.dot_general(pooled, wlin_ref[...], _DN, precision=_HI)
        logits = logits + blin_ref[...]
        mx = jnp.max(logits, axis=1, keepdims=True)
        e = jnp.exp(logits - mx)
        out_ref[...] = e / jnp.sum(e, axis=1, keepdims=True)


def _tc_final(agg_a, agg_b, r_prev, batch3, wlin, blin):
    return pl.pallas_call(
        _final_body,
        grid=(NBLK,),
        in_specs=[_ROW_SPEC, _ROW_SPEC, _ROW_SPEC,
                  pl.BlockSpec((1, 1, BLK), lambda i: (i, 0, 0)),
                  _W_SPEC, _B_SPEC],
        out_specs=pl.BlockSpec((G, D), lambda i: (0, 0)),
        out_shape=jax.ShapeDtypeStruct((G, D), jnp.float32),
        scratch_shapes=[pltpu.VMEM((G, D), jnp.float32),
                        pltpu.VMEM((G, D), jnp.float32)],
    )(agg_a, agg_b, r_prev, batch3, wlin, blin.reshape(1, D))


def _sc_segsum_body(y_hbm, src_hbm, dst_hbm, zeros_hbm, out_a, out_b,
                    src_cb, dst_b0, dst_b1, dst_b2, dst_b3, rows_v, agg_sh,
                    isem0, isem1, isem2, isem3,
                    jsem0, jsem1, jsem2, jsem3, gsem0, gsem1, semz):
    cid = lax.axis_index("c")
    sid = lax.axis_index("s")
    row0 = sid * ROWS_T

    zcopy = pltpu.async_copy(zeros_hbm.at[pl.ds(row0, ROWS_T)],
                             agg_sh.at[pl.ds(row0, ROWS_T)], semz)

    base = (cid * 16 + sid) * ETILE

    isems = (isem0, isem1, isem2, isem3)
    jsems = (jsem0, jsem1, jsem2, jsem3)
    gsems = (gsem0, gsem1)
    dstrefs = (dst_b0, dst_b1, dst_b2, dst_b3)

    def _fire_idx(c, sl):
        off = base + jnp.minimum(c, NCH - 1) * CHUNK
        pltpu.async_copy(src_hbm.at[pl.ds(off, CHUNK)], src_cb.at[sl],
                         isems[sl])
        pltpu.async_copy(dst_hbm.at[pl.ds(off, CHUNK)], dstrefs[sl],
                         jsems[sl])

    def _wait_idx(sl):
        pltpu.make_async_copy(src_hbm.at[pl.ds(base, CHUNK)], src_cb.at[sl],
                              isems[sl]).wait()
        pltpu.make_async_copy(dst_hbm.at[pl.ds(base, CHUNK)], dstrefs[sl],
                              jsems[sl]).wait()

    def _fire_gather(sl, rb):
        pltpu.async_copy(y_hbm.at[src_cb.at[sl]], rows_v.at[rb], gsems[rb])

    def _wait_gather(sl, rb):
        pltpu.make_async_copy(y_hbm.at[src_cb.at[sl]], rows_v.at[rb],
                              gsems[rb]).wait()

    for sl in range(4):
        _fire_idx(sl, sl)
    for c in range(2):
        _wait_idx(c)
        _fire_gather(c, c)
    zcopy.wait()
    plsc.subcore_barrier()

    def _body(p, carry):
        for u in range(4):
            c = 4 * p + u
            rb = u % 2
            _wait_gather(u, rb)
            pltpu.sync_copy(rows_v.at[rb], agg_sh.at[dstrefs[u]], add=True)
            _fire_idx(c + 4, u)
            _wait_idx((u + 2) % 4)
            _fire_gather((u + 2) % 4, rb)
        return carry

    lax.fori_loop(0, (NCH - 2) // 4, _body, 0)
    for u in range(2):
        rb = u % 2
        _wait_gather(u, rb)
        pltpu.sync_copy(rows_v.at[rb], agg_sh.at[dstrefs[u]], add=True)
        _wait_idx(u + 2)
    plsc.subcore_barrier()

    @pl.when(cid == 0)
    def _():
        pltpu.sync_copy(agg_sh.at[pl.ds(row0, ROWS_T)],
                        out_a.at[pl.ds(row0, ROWS_T)])

    @pl.when(cid == 1)
    def _():
        pltpu.sync_copy(agg_sh.at[pl.ds(row0, ROWS_T)],
                        out_b.at[pl.ds(row0, ROWS_T)])


@functools.lru_cache(maxsize=1)
def _get_sc_segsum():
    return pl.kernel(
        _sc_segsum_body,
        out_type=[jax.ShapeDtypeStruct((NP, D), jnp.float32)] * 2,
        mesh=plsc.VectorSubcoreMesh(core_axis_name="c", subcore_axis_name="s"),
        scratch_types=[
            pltpu.VMEM((4, CHUNK), jnp.int32),
            pltpu.VMEM((CHUNK,), jnp.int32),
            pltpu.VMEM((CHUNK,), jnp.int32),
            pltpu.VMEM((CHUNK,), jnp.int32),
            pltpu.VMEM((CHUNK,), jnp.int32),
            pltpu.VMEM((2, CHUNK, D), jnp.float32),
            pltpu.VMEM_SHARED((NP, D), jnp.float32),
        ] + [pltpu.SemaphoreType.DMA] * 11,
    )


def _sc_segsum(y, src, dst, zeros):
    return _get_sc_segsum()(y, src, dst, zeros)


def kernel(x, edge_index, batch,
           Wrel0, brel0, Wroot0,
           Wrel1, brel1, Wroot1,
           Wrel2, brel2, Wroot2,
           Wrel3, brel3, Wroot3,
           Wrel4, brel4, Wroot4,
           Wlin, blin):
    xp = jnp.zeros((NP, D), jnp.float32).at[:N].set(x)
    npad = EP - E
    pr = jnp.arange(npad, dtype=jnp.int32)
    src1 = jnp.concatenate([edge_index[0], pr * 13 % N])
    dst1 = jnp.concatenate([edge_index[1], N + pr % (NP - N)])
    zeros = jnp.zeros((NP, D), jnp.float32)
    batch3 = jnp.full((NP,), G, jnp.int32).at[:N].set(batch).reshape(NBLK, 1, BLK)

    y, r = _tc_stage_first(xp, Wrel0, Wroot0, brel0)
    for wrel, wroot, brel, add_h in (
            (Wrel1, Wroot1, brel1, False),
            (Wrel2, Wroot2, brel2, False),
            (Wrel3, Wroot3, brel3, True),
            (Wrel4, Wroot4, brel4, True)):
        agg_a, agg_b = _sc_segsum(y, src1, dst1, zeros)
        y, r = _tc_stage(agg_a, agg_b, r, wrel, wroot, brel, add_h)
    agg_a, agg_b = _sc_segsum(y, src1, dst1, zeros)
    return _tc_final(agg_a, agg_b, r, batch3, Wlin, blin)


# speedup vs baseline: 2.0074x; 1.0628x over previous
import functools

import jax
import jax.numpy as jnp
from jax import lax
from jax.experimental import pallas as pl
from jax.experimental.pallas import tpu as pltpu
from jax.experimental.pallas import tpu_sc as plsc

N = 10000
E = 320000
D = 128
G = 64

NP = 10240
BLK = 512
NBLK = NP // BLK

CHUNK = 128
EP = 335872
NCH = EP // 32 // CHUNK
ETILE = EP // 32
ROWS_T = NP // 16

_HI = lax.Precision.HIGHEST
_DN = (((1,), (1,)), ((), ()))
_DNT = (((1,), (0,)), ((), ()))


def _stage_first_body(h_ref, wrel_ref, wroot_ref, brel_ref, y_ref, r_ref):
    h = h_ref[...]
    y_ref[...] = lax.dot_general(h, wrel_ref[...], _DN, precision=_HI)
    r_ref[...] = lax.dot_general(h, wroot_ref[...], _DN, precision=_HI) + brel_ref[...]


def _stage_body(add_h, agga_ref, aggb_ref, rp_ref, wrel_ref, wroot_ref, brel_ref,
                y_ref, r_ref):
    h = jnp.maximum(agga_ref[...] + aggb_ref[...] + rp_ref[...], 0.0)
    y_ref[...] = lax.dot_general(h, wrel_ref[...], _DN, precision=_HI)
    r = lax.dot_general(h, wroot_ref[...], _DN, precision=_HI) + brel_ref[...]
    if add_h:
        r = r + h
    r_ref[...] = r


_ROW_SPEC = pl.BlockSpec((BLK, D), lambda i: (i, 0))
_W_SPEC = pl.BlockSpec((D, D), lambda i: (0, 0))
_B_SPEC = pl.BlockSpec((1, D), lambda i: (0, 0))


def _tc_stage_first(x, wrel, wroot, brel):
    return pl.pallas_call(
        _stage_first_body,
        grid=(NBLK,),
        in_specs=[_ROW_SPEC, _W_SPEC, _W_SPEC, _B_SPEC],
        out_specs=[_ROW_SPEC, _ROW_SPEC],
        out_shape=[jax.ShapeDtypeStruct((NP, D), jnp.float32)] * 2,
    )(x, wrel, wroot, brel.reshape(1, D))


def _tc_stage(agg_a, agg_b, r_prev, wrel, wroot, brel, add_h):
    return pl.pallas_call(
        functools.partial(_stage_body, add_h),
        grid=(NBLK,),
        in_specs=[_ROW_SPEC, _ROW_SPEC, _ROW_SPEC, _W_SPEC, _W_SPEC, _B_SPEC],
        out_specs=[_ROW_SPEC, _ROW_SPEC],
        out_shape=[jax.ShapeDtypeStruct((NP, D), jnp.float32)] * 2,
    )(agg_a, agg_b, r_prev, wrel, wroot, brel.reshape(1, D))


def _final_body(agga_ref, aggb_ref, rp_ref, batch_ref, wlin_ref, blin_ref,
                out_ref, sums_ref, cnts_ref):
    i = pl.program_id(0)

    @pl.when(i == 0)
    def _():
        sums_ref[...] = jnp.zeros_like(sums_ref)
        cnts_ref[...] = jnp.zeros_like(cnts_ref)

    h = jnp.maximum(agga_ref[...] + aggb_ref[...] + rp_ref[...], 0.0)
    b = batch_ref[0]
    gids = lax.broadcasted_iota(jnp.int32, (G, BLK), 0)
    m = (b == gids).astype(jnp.float32)
    sums_ref[...] += lax.dot_general(m, h, _DNT, precision=_HI)
    cnts_ref[...] += lax.dot_general(m, jnp.ones((BLK, D), jnp.float32), _DNT,
                                     precision=_HI)

    @pl.when(i == NBLK - 1)
    def _():
        pooled = sums_ref[...] / jnp.maximum(cnts_ref[...], 1.0)
        logits = lax.dot_general(pooled, wlin_ref[...], _DN, precision=_HI)
        logits = logits + blin_ref[...]
        mx = jnp.max(logits, axis=1, keepdims=True)
        e = jnp.exp(logits - mx)
        out_ref[...] = e / jnp.sum(e, axis=1, keepdims=True)


def _tc_final(agg_a, agg_b, r_prev, batch3, wlin, blin):
    return pl.pallas_call(
        _final_body,
        grid=(NBLK,),
        in_specs=[_ROW_SPEC, _ROW_SPEC, _ROW_SPEC,
                  pl.BlockSpec((1, 1, BLK), lambda i: (i, 0, 0)),
                  _W_SPEC, _B_SPEC],
        out_specs=pl.BlockSpec((G, D), lambda i: (0, 0)),
        out_shape=jax.ShapeDtypeStruct((G, D), jnp.float32),
        scratch_shapes=[pltpu.VMEM((G, D), jnp.float32),
                        pltpu.VMEM((G, D), jnp.float32)],
    )(agg_a, agg_b, r_prev, batch3, wlin, blin.reshape(1, D))


def _sc_segsum_body(y_hbm, src_hbm, dst_hbm, zeros_hbm, out_a, out_b,
                    src_cb, dst_b0, dst_b1, dst_b2, dst_b3, rows_v, agg_sh,
                    isem0, isem1, isem2, isem3,
                    jsem0, jsem1, jsem2, jsem3, gsem0, gsem1, semz):
    cid = lax.axis_index("c")
    sid = lax.axis_index("s")
    row0 = sid * ROWS_T

    zcopy = pltpu.async_copy(zeros_hbm.at[pl.ds(row0, ROWS_T)],
                             agg_sh.at[pl.ds(row0, ROWS_T)], semz)

    base = (cid * 16 + sid) * ETILE

    isems = (isem0, isem1, isem2, isem3)
    jsems = (jsem0, jsem1, jsem2, jsem3)
    gsems = (gsem0, gsem1)
    dstrefs = (dst_b0, dst_b1, dst_b2, dst_b3)

    def _fire_idx(c, sl):
        off = base + jnp.minimum(c, NCH - 1) * CHUNK
        pltpu.async_copy(src_hbm.at[pl.ds(off, CHUNK)], src_cb.at[sl],
                         isems[sl])
        pltpu.async_copy(dst_hbm.at[pl.ds(off, CHUNK)], dstrefs[sl],
                         jsems[sl])

    def _wait_idx(sl):
        pltpu.make_async_copy(src_hbm.at[pl.ds(base, CHUNK)], src_cb.at[sl],
                              isems[sl]).wait()
        pltpu.make_async_copy(dst_hbm.at[pl.ds(base, CHUNK)], dstrefs[sl],
                              jsems[sl]).wait()

    def _fire_gather(sl, rb):
        pltpu.async_copy(y_hbm.at[src_cb.at[sl]], rows_v.at[rb], gsems[rb])

    def _wait_gather(sl, rb):
        pltpu.make_async_copy(y_hbm.at[src_cb.at[sl]], rows_v.at[rb],
                              gsems[rb]).wait()

    for sl in range(4):
        _fire_idx(sl, sl)
    for c in range(2):
        _wait_idx(c)
        _fire_gather(c, c)
    zcopy.wait()
    plsc.subcore_barrier()

    def _body(p, carry):
        for u in range(4):
            c = 4 * p + u
            rb = u % 2
            _wait_gather(u, rb)
            pltpu.sync_copy(rows_v.at[rb], agg_sh.at[dstrefs[u]], add=True)
            _fire_idx(c + 4, u)
            _wait_idx((u + 2) % 4)
            _fire_gather((u + 2) % 4, rb)
        return carry

    lax.fori_loop(0, (NCH - 2) // 4, _body, 0)
    for u in range(2):
        rb = u % 2
        _wait_gather(u, rb)
        pltpu.sync_copy(rows_v.at[rb], agg_sh.at[dstrefs[u]], add=True)
        _wait_idx(u + 2)
    plsc.subcore_barrier()

    @pl.when(cid == 0)
    def _():
        pltpu.sync_copy(agg_sh.at[pl.ds(row0, ROWS_T)],
                        out_a.at[pl.ds(row0, ROWS_T)])

    @pl.when(cid == 1)
    def _():
        pltpu.sync_copy(agg_sh.at[pl.ds(row0, ROWS_T)],
                        out_b.at[pl.ds(row0, ROWS_T)])


@functools.lru_cache(maxsize=1)
def _get_sc_segsum():
    return pl.kernel(
        _sc_segsum_body,
        out_type=[jax.ShapeDtypeStruct((NP, D), jnp.float32)] * 2,
        mesh=plsc.VectorSubcoreMesh(core_axis_name="c", subcore_axis_name="s"),
        scratch_types=[
            pltpu.VMEM((4, CHUNK), jnp.int32),
            pltpu.VMEM((CHUNK,), jnp.int32),
            pltpu.VMEM((CHUNK,), jnp.int32),
            pltpu.VMEM((CHUNK,), jnp.int32),
            pltpu.VMEM((CHUNK,), jnp.int32),
            pltpu.VMEM((2, CHUNK, D), jnp.float32),
            pltpu.VMEM_SHARED((NP, D), jnp.float32),
        ] + [pltpu.SemaphoreType.DMA] * 11,
    )


def _sc_segsum(y, src, dst, zeros):
    return _get_sc_segsum()(y, src, dst, zeros)


def kernel(x, edge_index, batch,
           Wrel0, brel0, Wroot0,
           Wrel1, brel1, Wroot1,
           Wrel2, brel2, Wroot2,
           Wrel3, brel3, Wroot3,
           Wrel4, brel4, Wroot4,
           Wlin, blin):
    xp = jnp.zeros((NP, D), jnp.float32).at[:N].set(x)
    npad = EP - E
    pr = jnp.arange(npad, dtype=jnp.int32)
    src1 = jnp.concatenate([edge_index[0], pr * 13 % N])
    dst1 = jnp.concatenate([edge_index[1], N + pr % (NP - N)])
    zeros = jnp.zeros((NP, D), jnp.float32)
    batch3 = jnp.full((NP,), G, jnp.int32).at[:N].set(batch).reshape(NBLK, 1, BLK)

    y, r = _tc_stage_first(xp, Wrel0, Wroot0, brel0)
    for wrel, wroot, brel, add_h in (
            (Wrel1, Wroot1, brel1, False),
            (Wrel2, Wroot2, brel2, False),
            (Wrel3, Wroot3, brel3, True),
            (Wrel4, Wroot4, brel4, True)):
        agg_a, agg_b = _sc_segsum(y, src1, dst1, zeros)
        y, r = _tc_stage(agg_a, agg_b, r, wrel, wroot, brel, add_h)
    agg_a, agg_b = _sc_segsum(y, src1, dst1, zeros)
    return _tc_final(agg_a, agg_b, r, batch3, Wlin, blin)


# speedup vs baseline: 2.0257x; 1.0091x over previous
import functools

import jax
import jax.numpy as jnp
from jax import lax
from jax.experimental import pallas as pl
from jax.experimental.pallas import tpu as pltpu
from jax.experimental.pallas import tpu_sc as plsc

N = 10000
E = 320000
D = 128
G = 64

NP = 10240
BLK = 512
NBLK = NP // BLK

CHUNK = 128
EP = 335872
NCH = EP // 32 // CHUNK
ETILE = EP // 32
ROWS_T = NP // 16

_HI = lax.Precision.HIGHEST
_DN = (((1,), (1,)), ((), ()))
_DNT = (((1,), (0,)), ((), ()))


def _stage_first_y_body(h_ref, wrel_ref, y_ref):
    y_ref[...] = lax.dot_general(h_ref[...], wrel_ref[...], _DN, precision=_HI)


def _stage_first_r_body(h_ref, wroot_ref, brel_ref, r_ref):
    r_ref[...] = lax.dot_general(h_ref[...], wroot_ref[...], _DN,
                                 precision=_HI) + brel_ref[...]


def _stage_y_body(agga_ref, aggb_ref, rp_ref, wrel_ref, y_ref):
    h = jnp.maximum(agga_ref[...] + aggb_ref[...] + rp_ref[...], 0.0)
    y_ref[...] = lax.dot_general(h, wrel_ref[...], _DN, precision=_HI)


def _stage_r_body(add_h, agga_ref, aggb_ref, rp_ref, wroot_ref, brel_ref,
                  r_ref):
    h = jnp.maximum(agga_ref[...] + aggb_ref[...] + rp_ref[...], 0.0)
    r = lax.dot_general(h, wroot_ref[...], _DN, precision=_HI) + brel_ref[...]
    if add_h:
        r = r + h
    r_ref[...] = r


_ROW_SPEC = pl.BlockSpec((BLK, D), lambda i: (i, 0))
_W_SPEC = pl.BlockSpec((D, D), lambda i: (0, 0))
_B_SPEC = pl.BlockSpec((1, D), lambda i: (0, 0))


_OUT_NP = jax.ShapeDtypeStruct((NP, D), jnp.float32)


def _tc_stage_first_y(x, wrel):
    return pl.pallas_call(
        _stage_first_y_body, grid=(NBLK,),
        in_specs=[_ROW_SPEC, _W_SPEC],
        out_specs=_ROW_SPEC, out_shape=_OUT_NP,
    )(x, wrel)


def _tc_stage_first_r(x, wroot, brel):
    return pl.pallas_call(
        _stage_first_r_body, grid=(NBLK,),
        in_specs=[_ROW_SPEC, _W_SPEC, _B_SPEC],
        out_specs=_ROW_SPEC, out_shape=_OUT_NP,
    )(x, wroot, brel.reshape(1, D))


def _tc_stage_y(agg_a, agg_b, r_prev, wrel):
    return pl.pallas_call(
        _stage_y_body, grid=(NBLK,),
        in_specs=[_ROW_SPEC, _ROW_SPEC, _ROW_SPEC, _W_SPEC],
        out_specs=_ROW_SPEC, out_shape=_OUT_NP,
    )(agg_a, agg_b, r_prev, wrel)


def _tc_stage_r(agg_a, agg_b, r_prev, wroot, brel, add_h):
    return pl.pallas_call(
        functools.partial(_stage_r_body, add_h), grid=(NBLK,),
        in_specs=[_ROW_SPEC, _ROW_SPEC, _ROW_SPEC, _W_SPEC, _B_SPEC],
        out_specs=_ROW_SPEC, out_shape=_OUT_NP,
    )(agg_a, agg_b, r_prev, wroot, brel.reshape(1, D))


def _final_body(agga_ref, aggb_ref, rp_ref, batch_ref, wlin_ref, blin_ref,
                out_ref, sums_ref, cnts_ref):
    i = pl.program_id(0)

    @pl.when(i == 0)
    def _():
        sums_ref[...] = jnp.zeros_like(sums_ref)
        cnts_ref[...] = jnp.zeros_like(cnts_ref)

    h = jnp.maximum(agga_ref[...] + aggb_ref[...] + rp_ref[...], 0.0)
    b = batch_ref[0]
    gids = lax.broadcasted_iota(jnp.int32, (G, BLK), 0)
    m = (b == gids).astype(jnp.float32)
    sums_ref[...] += lax.dot_general(m, h, _DNT, precision=_HI)
    cnts_ref[...] += lax.dot_general(m, jnp.ones((BLK, D), jnp.float32), _DNT,
                                     precision=_HI)

    @pl.when(i == NBLK - 1)
    def _():
        pooled = sums_ref[...] / jnp.maximum(cnts_ref[...], 1.0)
        logits = lax.dot_general(pooled, wlin_ref[...], _DN, precision=_HI)
        logits = logits + blin_ref[...]
        mx = jnp.max(logits, axis=1, keepdims=True)
        e = jnp.exp(logits - mx)
        out_ref[...] = e / jnp.sum(e, axis=1, keepdims=True)


def _tc_final(agg_a, agg_b, r_prev, batch3, wlin, blin):
    return pl.pallas_call(
        _final_body,
        grid=(NBLK,),
        in_specs=[_ROW_SPEC, _ROW_SPEC, _ROW_SPEC,
                  pl.BlockSpec((1, 1, BLK), lambda i: (i, 0, 0)),
                  _W_SPEC, _B_SPEC],
        out_specs=pl.BlockSpec((G, D), lambda i: (0, 0)),
        out_shape=jax.ShapeDtypeStruct((G, D), jnp.float32),
        scratch_shapes=[pltpu.VMEM((G, D), jnp.float32),
                        pltpu.VMEM((G, D), jnp.float32)],
    )(agg_a, agg_b, r_prev, batch3, wlin, blin.reshape(1, D))


def _sc_segsum_body(y_hbm, src_hbm, dst_hbm, zeros_hbm, out_a, out_b,
                    src_cb, dst_b0, dst_b1, dst_b2, dst_b3, rows_v, agg_sh,
                    isem0, isem1, isem2, isem3,
                    jsem0, jsem1, jsem2, jsem3, gsem0, gsem1, semz):
    cid = lax.axis_index("c")
    sid = lax.axis_index("s")
    row0 = sid * ROWS_T

    zcopy = pltpu.async_copy(zeros_hbm.at[pl.ds(row0, ROWS_T)],
                             agg_sh.at[pl.ds(row0, ROWS_T)], semz)

    base = (cid * 16 + sid) * ETILE

    isems = (isem0, isem1, isem2, isem3)
    jsems = (jsem0, jsem1, jsem2, jsem3)
    gsems = (gsem0, gsem1)
    dstrefs = (dst_b0, dst_b1, dst_b2, dst_b3)

    def _fire_idx(c, sl):
        off = base + jnp.minimum(c, NCH - 1) * CHUNK
        pltpu.async_copy(src_hbm.at[pl.ds(off, CHUNK)], src_cb.at[sl],
                         isems[sl])
        pltpu.async_copy(dst_hbm.at[pl.ds(off, CHUNK)], dstrefs[sl],
                         jsems[sl])

    def _wait_idx(sl):
        pltpu.make_async_copy(src_hbm.at[pl.ds(base, CHUNK)], src_cb.at[sl],
                              isems[sl]).wait()
        pltpu.make_async_copy(dst_hbm.at[pl.ds(base, CHUNK)], dstrefs[sl],
                              jsems[sl]).wait()

    def _fire_gather(sl, rb):
        pltpu.async_copy(y_hbm.at[src_cb.at[sl]], rows_v.at[rb], gsems[rb])

    def _wait_gather(sl, rb):
        pltpu.make_async_copy(y_hbm.at[src_cb.at[sl]], rows_v.at[rb],
                              gsems[rb]).wait()

    for sl in range(4):
        _fire_idx(sl, sl)
    for c in range(2):
        _wait_idx(c)
        _fire_gather(c, c)
    zcopy.wait()
    plsc.subcore_barrier()

    def _body(p, carry):
        for u in range(4):
            c = 4 * p + u
            rb = u % 2
            _wait_gather(u, rb)
            pltpu.sync_copy(rows_v.at[rb], agg_sh.at[dstrefs[u]], add=True)
            _fire_idx(c + 4, u)
            _wait_idx((u + 2) % 4)
            _fire_gather((u + 2) % 4, rb)
        return carry

    lax.fori_loop(0, (NCH - 2) // 4, _body, 0)
    for u in range(2):
        rb = u % 2
        _wait_gather(u, rb)
        pltpu.sync_copy(rows_v.at[rb], agg_sh.at[dstrefs[u]], add=True)
        _wait_idx(u + 2)
    plsc.subcore_barrier()

    @pl.when(cid == 0)
    def _():
        pltpu.sync_copy(agg_sh.at[pl.ds(row0, ROWS_T)],
                        out_a.at[pl.ds(row0, ROWS_T)])

    @pl.when(cid == 1)
    def _():
        pltpu.sync_copy(agg_sh.at[pl.ds(row0, ROWS_T)],
                        out_b.at[pl.ds(row0, ROWS_T)])


@functools.lru_cache(maxsize=1)
def _get_sc_segsum():
    return pl.kernel(
        _sc_segsum_body,
        out_type=[jax.ShapeDtypeStruct((NP, D), jnp.float32)] * 2,
        mesh=plsc.VectorSubcoreMesh(core_axis_name="c", subcore_axis_name="s"),
        scratch_types=[
            pltpu.VMEM((4, CHUNK), jnp.int32),
            pltpu.VMEM((CHUNK,), jnp.int32),
            pltpu.VMEM((CHUNK,), jnp.int32),
            pltpu.VMEM((CHUNK,), jnp.int32),
            pltpu.VMEM((CHUNK,), jnp.int32),
            pltpu.VMEM((2, CHUNK, D), jnp.float32),
            pltpu.VMEM_SHARED((NP, D), jnp.float32),
        ] + [pltpu.SemaphoreType.DMA] * 11,
    )


def _sc_segsum(y, src, dst, zeros):
    return _get_sc_segsum()(y, src, dst, zeros)


def kernel(x, edge_index, batch,
           Wrel0, brel0, Wroot0,
           Wrel1, brel1, Wroot1,
           Wrel2, brel2, Wroot2,
           Wrel3, brel3, Wroot3,
           Wrel4, brel4, Wroot4,
           Wlin, blin):
    xp = jnp.zeros((NP, D), jnp.float32).at[:N].set(x)
    npad = EP - E
    pr = jnp.arange(npad, dtype=jnp.int32)
    src1 = jnp.concatenate([edge_index[0], pr * 13 % N])
    dst1 = jnp.concatenate([edge_index[1], N + pr % (NP - N)])
    zeros = jnp.zeros((NP, D), jnp.float32)
    batch3 = jnp.full((NP,), G, jnp.int32).at[:N].set(batch).reshape(NBLK, 1, BLK)

    y = _tc_stage_first_y(xp, Wrel0)
    r = _tc_stage_first_r(xp, Wroot0, brel0)
    for wrel, wroot, brel, add_h in (
            (Wrel1, Wroot1, brel1, False),
            (Wrel2, Wroot2, brel2, False),
            (Wrel3, Wroot3, brel3, True),
            (Wrel4, Wroot4, brel4, True)):
        agg_a, agg_b = _sc_segsum(y, src1, dst1, zeros)
        y = _tc_stage_y(agg_a, agg_b, r, wrel)
        r = _tc_stage_r(agg_a, agg_b, r, wroot, brel, add_h)
    agg_a, agg_b = _sc_segsum(y, src1, dst1, zeros)
    return _tc_final(agg_a, agg_b, r, batch3, Wlin, blin)
